# Initial kernel scaffold; baseline (speedup 1.0000x reference)
import jax
import jax.numpy as jnp
from jax.experimental import pallas as pl


def kernel(q, pos_embed):
    raise NotImplementedError("write your pallas kernel here")


# speedup vs baseline: 2.0125x; 2.0125x over previous
import jax
import jax.numpy as jnp
from jax.experimental import pallas as pl
from jax.experimental.pallas import tpu as pltpu

_TB = 512


def _body(q_ref, pos_ref, out_ref):
    out_ref[...] = q_ref[...] + jnp.swapaxes(pos_ref[...], 0, 1)[None]


def kernel(q, pos_embed):
    bsz, d_model, q_frm = q.shape
    grid = (q_frm // _TB,)
    return pl.pallas_call(
        _body,
        grid=grid,
        in_specs=[
            pl.BlockSpec((bsz, d_model, _TB), lambda t: (0, 0, t)),
            pl.BlockSpec((_TB, d_model), lambda t: (t, 0)),
        ],
        out_specs=pl.BlockSpec((bsz, d_model, _TB), lambda t: (0, 0, t)),
        out_shape=jax.ShapeDtypeStruct((bsz, d_model, q_frm), q.dtype),
        compiler_params=pltpu.CompilerParams(
            dimension_semantics=("arbitrary",),
        ),
    )(q, pos_embed)


# speedup vs baseline: 2.0393x; 1.0133x over previous
import jax
import jax.numpy as jnp
from jax.experimental import pallas as pl
from jax.experimental.pallas import tpu as pltpu

_TB = 256


def _body(q_ref, pos_ref, out_ref):
    out_ref[...] = q_ref[...] + jnp.swapaxes(pos_ref[...], 0, 1)[None]


def kernel(q, pos_embed):
    bsz, d_model, q_frm = q.shape
    grid = (q_frm // _TB,)
    return pl.pallas_call(
        _body,
        grid=grid,
        in_specs=[
            pl.BlockSpec((bsz, d_model, _TB), lambda t: (0, 0, t)),
            pl.BlockSpec((_TB, d_model), lambda t: (t, 0)),
        ],
        out_specs=pl.BlockSpec((bsz, d_model, _TB), lambda t: (0, 0, t)),
        out_shape=jax.ShapeDtypeStruct((bsz, d_model, q_frm), q.dtype),
        compiler_params=pltpu.CompilerParams(
            dimension_semantics=("arbitrary",),
        ),
    )(q, pos_embed)
